# baseline (speedup 1.0000x reference)
import functools

import jax
import jax.numpy as jnp
from jax import lax
from jax.experimental import pallas as pl
from jax.experimental.pallas import tpu as pltpu
from jax.experimental.pallas import tpu_sc as plsc

N = 10000
E = 320000
D_IN = 128
D_HID = 256
D_OUT = 128
EPS = 1e-5

DH = 64
NPAD = 10240
EPAD = 327680
CH = 512
NPW = NPAD // 16
EPW_A = EPAD // 16
EPW_D = EPAD // 32

_mesh = plsc.VectorSubcoreMesh(core_axis_name="c", subcore_axis_name="s")


@functools.partial(
    pl.kernel,
    out_type=jax.ShapeDtypeStruct((2, NPAD), jnp.float32),
    mesh=_mesh,
    scratch_types=[
        pltpu.VMEM((4, 128), jnp.int32),
        pltpu.VMEM((128,), jnp.float32),
        pltpu.VMEM((NPW,), jnp.float32),
        pltpu.VMEM_SHARED((NPAD,), jnp.float32),
    ],
)
def _deg_sc(dst2d, out, idx_v, ones_v, zer_v, acc):
    cid = lax.axis_index("c")
    sid = lax.axis_index("s")
    wid = cid * 16 + sid
    for i in range(8):
        ones_v[pl.ds(i * 16, 16)] = jnp.ones((16,), jnp.float32)

    def zero_body(i, c):
        zer_v[pl.ds(i * 16, 16)] = jnp.zeros((16,), jnp.float32)
        return c

    lax.fori_loop(0, NPW // 16, zero_body, 0)
    pltpu.sync_copy(zer_v, acc.at[pl.ds(sid * NPW, NPW)])
    plsc.subcore_barrier()

    base = wid * (EPW_D // 128)

    def body(ci, c):
        pltpu.sync_copy(dst2d.at[pl.ds(base + ci * 4, 4)], idx_v)
        for j in range(4):
            pltpu.sync_copy(ones_v, acc.at[idx_v.at[j]], add=True)
        return c

    lax.fori_loop(0, EPW_D // CH, body, 0)
    plsc.subcore_barrier()

    sl = pl.ds(sid * NPW, NPW)
    pltpu.sync_copy(acc.at[sl], out.at[cid, sl])


NB = EPW_A // 128
GS = 2
NGG = NB // GS
IRD = 4


@functools.partial(
    pl.kernel,
    out_type=jax.ShapeDtypeStruct((2, NPAD, DH), jnp.float32),
    mesh=_mesh,
    scratch_types=[
        pltpu.VMEM((IRD, GS, 128), jnp.int32),
        pltpu.VMEM((IRD, GS, 128), jnp.int32),
        pltpu.VMEM((2 * GS, 128, DH), jnp.float32),
        pltpu.VMEM_SHARED((NPAD, DH), jnp.float32),
        pltpu.VMEM_SHARED((NPAD, DH), jnp.float32),
        pltpu.SemaphoreType.DMA,
        pltpu.SemaphoreType.DMA,
        pltpu.SemaphoreType.DMA,
    ],
    compiler_params=pltpu.CompilerParams(use_tc_tiling_on_sc=False),
)
def _agg_sc(feat, src2d, dst2d, out, sring, dring, rows, acc, feat_s,
            gsem, ssem, isem):
    cid = lax.axis_index("c")
    sid = lax.axis_index("s")

    def zero_body(i, c):
        for j in range(DH // 16):
            rows[0, i, pl.ds(j * 16, 16)] = jnp.zeros((16,), jnp.float32)
        return c

    lax.fori_loop(0, 128, zero_body, 0)
    for t in range(NPW // 128):
        pltpu.sync_copy(rows.at[0], acc.at[pl.ds(sid * NPW + t * 128, 128)])
    pltpu.sync_copy(feat.at[pl.ds(cid * NPAD + sid * NPW, NPW)],
                    feat_s.at[pl.ds(sid * NPW, NPW)])
    plsc.subcore_barrier()

    base = sid * NB

    def fire_i(g):
        r = g % IRD
        pltpu.async_copy(src2d.at[pl.ds(base + g * GS, GS)], sring.at[r],
                         isem)
        pltpu.async_copy(dst2d.at[pl.ds(base + g * GS, GS)], dring.at[r],
                         isem)

    def drain_i(g):
        r = g % IRD
        pltpu.make_async_copy(src2d.at[pl.ds(base, GS)], sring.at[r],
                              isem).wait()
        pltpu.make_async_copy(dst2d.at[pl.ds(base, GS)], dring.at[r],
                              isem).wait()

    def fire_g(g):
        r = g % IRD
        sb = (g % 2) * GS
        for j in range(GS):
            pltpu.async_copy(feat_s.at[sring.at[r, j]], rows.at[sb + j],
                             gsem)

    def fire_s(g):
        r = g % IRD
        sb = (g % 2) * GS
        for j in range(GS):
            pltpu.async_copy(rows.at[sb + j], acc.at[dring.at[r, j]],
                             ssem, add=True)

    def drain_g(g):
        sb = (g % 2) * GS
        for j in range(GS):
            pltpu.make_async_copy(feat_s.at[sring.at[0, 0]], rows.at[sb + j],
                                  gsem).wait()

    def drain_s(g):
        sb = (g % 2) * GS
        for j in range(GS):
            pltpu.make_async_copy(rows.at[sb + j], acc.at[dring.at[0, 0]],
                                  ssem).wait()

    fire_i(0)
    fire_i(1)
    fire_i(2)
    drain_i(0)
    fire_g(0)
    drain_i(1)
    drain_g(0)
    fire_g(1)
    fire_s(0)
    fire_i(3)
    drain_i(2)

    def body(g, c):
        drain_g(g)
        drain_s(g - 1)
        drain_i(g + 2)
        fire_g(g + 1)
        fire_s(g)
        fire_i(g + 3)
        return c

    lax.fori_loop(1, NGG - 3, body, 0)

    g = NGG - 3
    drain_g(g)
    drain_s(g - 1)
    drain_i(g + 2)
    fire_g(g + 1)
    fire_s(g)
    g = NGG - 2
    drain_g(g)
    drain_s(g - 1)
    fire_g(g + 1)
    fire_s(g)
    g = NGG - 1
    drain_g(g)
    drain_s(g - 1)
    fire_s(g)
    drain_s(g)

    plsc.subcore_barrier()

    for t in range(NPW // 128):
        sl = pl.ds(sid * NPW + t * 128, 128)
        pltpu.sync_copy(acc.at[sl], out.at[cid, sl])


BR = 1000
GRID = N // BR


def _prep_body(d0_ref, d1_ref, x_ref, xs_ref, dinv_ref):
    dinv = lax.rsqrt(d0_ref[0] + d1_ref[0] + 1.0)
    dinv_ref[...] = dinv
    xs = x_ref[...] * dinv
    xs_ref[0] = xs[:, :DH]
    xs_ref[1] = xs[:, DH:]


_prep = pl.pallas_call(
    _prep_body,
    grid=(GRID,),
    in_specs=[pl.BlockSpec((1, BR, 1), lambda i: (0, i, 0)),
              pl.BlockSpec((1, BR, 1), lambda i: (1, i, 0)),
              pl.BlockSpec((BR, D_IN), lambda i: (i, 0))],
    out_specs=[pl.BlockSpec((2, BR, DH), lambda i: (0, i, 0)),
               pl.BlockSpec((BR, 1), lambda i: (i, 0))],
    out_shape=[jax.ShapeDtypeStruct((2, NPAD, DH), jnp.float32),
               jax.ShapeDtypeStruct((NPAD, 1), jnp.float32)],
)


def _l1_body(dinv_ref, p0_ref, p1_ref, xs0_ref, xs1_ref, w1_ref, b1_ref,
             gw_ref, gb_ref, gms_ref, w2_ref, z_ref, hscr, acc1, acc2):
    p = pl.program_id(0)
    i = pl.program_id(1)

    @pl.when(p == 0)
    def _():
        a = jnp.concatenate([p0_ref[0] + xs0_ref[0], p1_ref[0] + xs1_ref[0]],
                            axis=1) * dinv_ref[...]
        h = (jnp.dot(a, w1_ref[...], preferred_element_type=jnp.float32)
             + b1_ref[...])
        hscr[pl.ds(i * BR, BR), :] = h

        @pl.when(i == 0)
        def _():
            acc1[...] = jnp.zeros_like(acc1)
            acc2[...] = jnp.zeros_like(acc2)

        acc1[...] += jnp.sum(h, axis=0, keepdims=True)
        acc2[...] += jnp.sum(h * h, axis=0, keepdims=True)

    @pl.when(p == 1)
    def _():
        s1 = acc1[...] * (1.0 / N)
        mms = s1 * gms_ref[...]
        var = acc2[...] * (1.0 / N) - 2.0 * mms * s1 + mms * mms
        scale = gw_ref[...] * lax.rsqrt(var + EPS)
        h = hscr[pl.ds(i * BR, BR), :]
        g = jnp.maximum(scale * (h - mms) + gb_ref[...], 0.0)
        z = jnp.dot(g, w2_ref[...], preferred_element_type=jnp.float32)
        z = z * dinv_ref[...]
        z_ref[0] = z[:, :DH]
        z_ref[1] = z[:, DH:]


_layer12 = pl.pallas_call(
    _l1_body,
    grid=(2, GRID),
    in_specs=[pl.BlockSpec((BR, 1), lambda p, i: (i, 0)),
              pl.BlockSpec((1, BR, DH), lambda p, i: (0, i, 0)),
              pl.BlockSpec((1, BR, DH), lambda p, i: (1, i, 0)),
              pl.BlockSpec((1, BR, DH), lambda p, i: (0, i, 0)),
              pl.BlockSpec((1, BR, DH), lambda p, i: (1, i, 0)),
              pl.BlockSpec((D_IN, D_HID), lambda p, i: (0, 0)),
              pl.BlockSpec((1, D_HID), lambda p, i: (0, 0)),
              pl.BlockSpec((1, D_HID), lambda p, i: (0, 0)),
              pl.BlockSpec((1, D_HID), lambda p, i: (0, 0)),
              pl.BlockSpec((1, D_HID), lambda p, i: (0, 0)),
              pl.BlockSpec((D_HID, D_OUT), lambda p, i: (0, 0))],
    out_specs=pl.BlockSpec((2, BR, DH), lambda p, i: (0, i, 0)),
    out_shape=jax.ShapeDtypeStruct((2, NPAD, DH), jnp.float32),
    scratch_shapes=[pltpu.VMEM((N, D_HID), jnp.float32),
                    pltpu.VMEM((1, D_HID), jnp.float32),
                    pltpu.VMEM((1, D_HID), jnp.float32)],
)


def _fin_body(dinv_ref, q0_ref, q1_ref, zs0_ref, zs1_ref, b2_ref,
              gw_ref, gb_ref, gms_ref, o_ref, hscr, acc1, acc2):
    p = pl.program_id(0)
    i = pl.program_id(1)

    @pl.when(p == 0)
    def _():
        h = (jnp.concatenate([q0_ref[0] + zs0_ref[0], q1_ref[0] + zs1_ref[0]],
                             axis=1) * dinv_ref[...] + b2_ref[...])
        hscr[pl.ds(i * BR, BR), :] = h

        @pl.when(i == 0)
        def _():
            acc1[...] = jnp.zeros_like(acc1)
            acc2[...] = jnp.zeros_like(acc2)

        acc1[...] += jnp.sum(h, axis=0, keepdims=True)
        acc2[...] += jnp.sum(h * h, axis=0, keepdims=True)

    @pl.when(p == 1)
    def _():
        s1 = acc1[...] * (1.0 / N)
        mms = s1 * gms_ref[...]
        var = acc2[...] * (1.0 / N) - 2.0 * mms * s1 + mms * mms
        scale = gw_ref[...] * lax.rsqrt(var + EPS)
        h = hscr[pl.ds(i * BR, BR), :]
        o_ref[...] = jnp.maximum(scale * (h - mms) + gb_ref[...], 0.0)


_final = pl.pallas_call(
    _fin_body,
    grid=(2, GRID),
    in_specs=[pl.BlockSpec((BR, 1), lambda p, i: (i, 0)),
              pl.BlockSpec((1, BR, DH), lambda p, i: (0, i, 0)),
              pl.BlockSpec((1, BR, DH), lambda p, i: (1, i, 0)),
              pl.BlockSpec((1, BR, DH), lambda p, i: (0, i, 0)),
              pl.BlockSpec((1, BR, DH), lambda p, i: (1, i, 0)),
              pl.BlockSpec((1, D_OUT), lambda p, i: (0, 0)),
              pl.BlockSpec((1, D_OUT), lambda p, i: (0, 0)),
              pl.BlockSpec((1, D_OUT), lambda p, i: (0, 0)),
              pl.BlockSpec((1, D_OUT), lambda p, i: (0, 0))],
    out_specs=pl.BlockSpec((BR, D_OUT), lambda p, i: (i, 0)),
    out_shape=jax.ShapeDtypeStruct((N, D_OUT), jnp.float32),
    scratch_shapes=[pltpu.VMEM((N, D_OUT), jnp.float32),
                    pltpu.VMEM((1, D_OUT), jnp.float32),
                    pltpu.VMEM((1, D_OUT), jnp.float32)],
)


def kernel(x, edge_index, W1, b1, gn1_w, gn1_b, gn1_ms, W2, b2,
           gn2_w, gn2_b, gn2_ms):
    pad = EPAD - E
    src2d = jnp.concatenate(
        [edge_index[0], jnp.zeros((pad,), jnp.int32)]).reshape(-1, 128)
    dst2d = jnp.concatenate(
        [edge_index[1], jnp.full((pad,), NPAD - 1, jnp.int32)]).reshape(-1, 128)

    deg = _deg_sc(dst2d).reshape(2, NPAD, 1)
    xs3, dinv = _prep(deg, deg, x)
    p = _agg_sc(xs3.reshape(2 * NPAD, DH), src2d, dst2d)
    zs3 = _layer12(dinv, p, p, xs3, xs3, W1, b1.reshape(1, D_HID),
                   gn1_w.reshape(1, D_HID), gn1_b.reshape(1, D_HID),
                   gn1_ms.reshape(1, D_HID), W2)
    q = _agg_sc(zs3.reshape(2 * NPAD, DH), src2d, dst2d)
    return _final(dinv, q, q, zs3, zs3, b2.reshape(1, D_OUT),
                  gn2_w.reshape(1, D_OUT), gn2_b.reshape(1, D_OUT),
                  gn2_ms.reshape(1, D_OUT))


# speedup vs baseline: 1.0195x; 1.0195x over previous
import functools

import jax
import jax.numpy as jnp
from jax import lax
from jax.experimental import pallas as pl
from jax.experimental.pallas import tpu as pltpu
from jax.experimental.pallas import tpu_sc as plsc

N = 10000
E = 320000
D_IN = 128
D_HID = 256
D_OUT = 128
EPS = 1e-5

DH = 64
NPAD = 10240
EPAD = 327680
CH = 512
NPW = NPAD // 16
EPW_A = EPAD // 16
EPW_D = EPAD // 32

_mesh = plsc.VectorSubcoreMesh(core_axis_name="c", subcore_axis_name="s")


NCHD = EPW_D // CH


@functools.partial(
    pl.kernel,
    out_type=jax.ShapeDtypeStruct((2, NPAD), jnp.float32),
    mesh=_mesh,
    scratch_types=[
        pltpu.VMEM((2, 4, 128), jnp.int32),
        pltpu.VMEM((128,), jnp.float32),
        pltpu.VMEM((NPW,), jnp.float32),
        pltpu.VMEM_SHARED((NPAD,), jnp.float32),
        pltpu.SemaphoreType.DMA,
        pltpu.SemaphoreType.DMA,
    ],
)
def _deg_sc(dst2d, out, dring, ones_v, zer_v, acc, ssem, isem):
    cid = lax.axis_index("c")
    sid = lax.axis_index("s")
    wid = cid * 16 + sid
    for i in range(8):
        ones_v[pl.ds(i * 16, 16)] = jnp.ones((16,), jnp.float32)

    def zero_body(i, c):
        zer_v[pl.ds(i * 16, 16)] = jnp.zeros((16,), jnp.float32)
        return c

    lax.fori_loop(0, NPW // 16, zero_body, 0)
    pltpu.sync_copy(zer_v, acc.at[pl.ds(sid * NPW, NPW)])
    plsc.subcore_barrier()

    base = wid * (EPW_D // 128)

    def fire_i(ci):
        pltpu.async_copy(dst2d.at[pl.ds(base + ci * 4, 4)],
                         dring.at[ci % 2], isem)

    def drain_i(ci):
        pltpu.make_async_copy(dst2d.at[pl.ds(base, 4)], dring.at[ci % 2],
                              isem).wait()

    def fire_s(ci):
        for j in range(4):
            pltpu.async_copy(ones_v, acc.at[dring.at[ci % 2, j]], ssem,
                             add=True)

    def drain_s(ci):
        for j in range(4):
            pltpu.make_async_copy(ones_v, acc.at[dring.at[0, 0]],
                                  ssem).wait()

    fire_i(0)
    fire_i(1)
    drain_i(0)
    fire_s(0)

    def body(ci, c):
        drain_s(ci - 1)
        fire_i(ci + 1)
        drain_i(ci)
        fire_s(ci)
        return c

    lax.fori_loop(1, NCHD - 1, body, 0)
    drain_s(NCHD - 2)
    drain_i(NCHD - 1)
    fire_s(NCHD - 1)
    drain_s(NCHD - 1)
    plsc.subcore_barrier()

    sl = pl.ds(sid * NPW, NPW)
    pltpu.sync_copy(acc.at[sl], out.at[cid, sl])


NB = EPW_A // 128
GS = 2
NGG = NB // GS
IRD = 4


@functools.partial(
    pl.kernel,
    out_type=jax.ShapeDtypeStruct((2, NPAD, DH), jnp.float32),
    mesh=_mesh,
    scratch_types=[
        pltpu.VMEM((IRD, GS, 128), jnp.int32),
        pltpu.VMEM((IRD, GS, 128), jnp.int32),
        pltpu.VMEM((2 * GS, 128, DH), jnp.float32),
        pltpu.VMEM_SHARED((NPAD, DH), jnp.float32),
        pltpu.VMEM_SHARED((NPAD, DH), jnp.float32),
        pltpu.SemaphoreType.DMA,
        pltpu.SemaphoreType.DMA,
        pltpu.SemaphoreType.DMA,
    ],
    compiler_params=pltpu.CompilerParams(use_tc_tiling_on_sc=False),
)
def _agg_sc(feat, src2d, dst2d, out, sring, dring, rows, acc, feat_s,
            gsem, ssem, isem):
    cid = lax.axis_index("c")
    sid = lax.axis_index("s")

    def zero_body(i, c):
        for j in range(DH // 16):
            rows[0, i, pl.ds(j * 16, 16)] = jnp.zeros((16,), jnp.float32)
        return c

    lax.fori_loop(0, 128, zero_body, 0)
    for t in range(NPW // 128):
        pltpu.sync_copy(rows.at[0], acc.at[pl.ds(sid * NPW + t * 128, 128)])
    pltpu.sync_copy(feat.at[pl.ds(cid * NPAD + sid * NPW, NPW)],
                    feat_s.at[pl.ds(sid * NPW, NPW)])
    plsc.subcore_barrier()

    base = sid * NB

    def fire_i(g):
        r = g % IRD
        pltpu.async_copy(src2d.at[pl.ds(base + g * GS, GS)], sring.at[r],
                         isem)
        pltpu.async_copy(dst2d.at[pl.ds(base + g * GS, GS)], dring.at[r],
                         isem)

    def drain_i(g):
        r = g % IRD
        pltpu.make_async_copy(src2d.at[pl.ds(base, GS)], sring.at[r],
                              isem).wait()
        pltpu.make_async_copy(dst2d.at[pl.ds(base, GS)], dring.at[r],
                              isem).wait()

    def fire_g(g):
        r = g % IRD
        sb = (g % 2) * GS
        for j in range(GS):
            pltpu.async_copy(feat_s.at[sring.at[r, j]], rows.at[sb + j],
                             gsem)

    def fire_s(g):
        r = g % IRD
        sb = (g % 2) * GS
        for j in range(GS):
            pltpu.async_copy(rows.at[sb + j], acc.at[dring.at[r, j]],
                             ssem, add=True)

    def drain_g(g):
        sb = (g % 2) * GS
        for j in range(GS):
            pltpu.make_async_copy(feat_s.at[sring.at[0, 0]], rows.at[sb + j],
                                  gsem).wait()

    def drain_s(g):
        sb = (g % 2) * GS
        for j in range(GS):
            pltpu.make_async_copy(rows.at[sb + j], acc.at[dring.at[0, 0]],
                                  ssem).wait()

    fire_i(0)
    fire_i(1)
    fire_i(2)
    drain_i(0)
    fire_g(0)
    drain_i(1)
    drain_g(0)
    fire_g(1)
    fire_s(0)
    fire_i(3)
    drain_i(2)

    def body(g, c):
        drain_g(g)
        drain_s(g - 1)
        drain_i(g + 2)
        fire_g(g + 1)
        fire_s(g)
        fire_i(g + 3)
        return c

    lax.fori_loop(1, NGG - 3, body, 0)

    g = NGG - 3
    drain_g(g)
    drain_s(g - 1)
    drain_i(g + 2)
    fire_g(g + 1)
    fire_s(g)
    g = NGG - 2
    drain_g(g)
    drain_s(g - 1)
    fire_g(g + 1)
    fire_s(g)
    g = NGG - 1
    drain_g(g)
    drain_s(g - 1)
    fire_s(g)
    drain_s(g)

    plsc.subcore_barrier()

    for t in range(NPW // 128):
        sl = pl.ds(sid * NPW + t * 128, 128)
        pltpu.sync_copy(acc.at[sl], out.at[cid, sl])


BR = 1000
GRID = N // BR


def _prep_body(d0_ref, d1_ref, x_ref, xs_ref, dinv_ref):
    dinv = lax.rsqrt(d0_ref[0] + d1_ref[0] + 1.0)
    dinv_ref[...] = dinv
    xs = x_ref[...] * dinv
    xs_ref[0] = xs[:, :DH]
    xs_ref[1] = xs[:, DH:]


_prep = pl.pallas_call(
    _prep_body,
    grid=(GRID,),
    in_specs=[pl.BlockSpec((1, BR, 1), lambda i: (0, i, 0)),
              pl.BlockSpec((1, BR, 1), lambda i: (1, i, 0)),
              pl.BlockSpec((BR, D_IN), lambda i: (i, 0))],
    out_specs=[pl.BlockSpec((2, BR, DH), lambda i: (0, i, 0)),
               pl.BlockSpec((BR, 1), lambda i: (i, 0))],
    out_shape=[jax.ShapeDtypeStruct((2, NPAD, DH), jnp.float32),
               jax.ShapeDtypeStruct((NPAD, 1), jnp.float32)],
)


def _l1_body(dinv_ref, p0_ref, p1_ref, xs0_ref, xs1_ref, w1_ref, b1_ref,
             gw_ref, gb_ref, gms_ref, w2_ref, z_ref, hscr, acc1, acc2):
    p = pl.program_id(0)
    i = pl.program_id(1)

    @pl.when(p == 0)
    def _():
        a = jnp.concatenate([p0_ref[0] + xs0_ref[0], p1_ref[0] + xs1_ref[0]],
                            axis=1) * dinv_ref[...]
        h = (jnp.dot(a, w1_ref[...], preferred_element_type=jnp.float32)
             + b1_ref[...])
        hscr[pl.ds(i * BR, BR), :] = h

        @pl.when(i == 0)
        def _():
            acc1[...] = jnp.zeros_like(acc1)
            acc2[...] = jnp.zeros_like(acc2)

        acc1[...] += jnp.sum(h, axis=0, keepdims=True)
        acc2[...] += jnp.sum(h * h, axis=0, keepdims=True)

    @pl.when(p == 1)
    def _():
        s1 = acc1[...] * (1.0 / N)
        mms = s1 * gms_ref[...]
        var = acc2[...] * (1.0 / N) - 2.0 * mms * s1 + mms * mms
        scale = gw_ref[...] * lax.rsqrt(var + EPS)
        h = hscr[pl.ds(i * BR, BR), :]
        g = jnp.maximum(scale * (h - mms) + gb_ref[...], 0.0)
        z = jnp.dot(g, w2_ref[...], preferred_element_type=jnp.float32)
        z = z * dinv_ref[...]
        z_ref[0] = z[:, :DH]
        z_ref[1] = z[:, DH:]


_layer12 = pl.pallas_call(
    _l1_body,
    grid=(2, GRID),
    in_specs=[pl.BlockSpec((BR, 1), lambda p, i: (i, 0)),
              pl.BlockSpec((1, BR, DH), lambda p, i: (0, i, 0)),
              pl.BlockSpec((1, BR, DH), lambda p, i: (1, i, 0)),
              pl.BlockSpec((1, BR, DH), lambda p, i: (0, i, 0)),
              pl.BlockSpec((1, BR, DH), lambda p, i: (1, i, 0)),
              pl.BlockSpec((D_IN, D_HID), lambda p, i: (0, 0)),
              pl.BlockSpec((1, D_HID), lambda p, i: (0, 0)),
              pl.BlockSpec((1, D_HID), lambda p, i: (0, 0)),
              pl.BlockSpec((1, D_HID), lambda p, i: (0, 0)),
              pl.BlockSpec((1, D_HID), lambda p, i: (0, 0)),
              pl.BlockSpec((D_HID, D_OUT), lambda p, i: (0, 0))],
    out_specs=pl.BlockSpec((2, BR, DH), lambda p, i: (0, i, 0)),
    out_shape=jax.ShapeDtypeStruct((2, NPAD, DH), jnp.float32),
    scratch_shapes=[pltpu.VMEM((N, D_HID), jnp.float32),
                    pltpu.VMEM((1, D_HID), jnp.float32),
                    pltpu.VMEM((1, D_HID), jnp.float32)],
)


def _fin_body(dinv_ref, q0_ref, q1_ref, zs0_ref, zs1_ref, b2_ref,
              gw_ref, gb_ref, gms_ref, o_ref, hscr, acc1, acc2):
    p = pl.program_id(0)
    i = pl.program_id(1)

    @pl.when(p == 0)
    def _():
        h = (jnp.concatenate([q0_ref[0] + zs0_ref[0], q1_ref[0] + zs1_ref[0]],
                             axis=1) * dinv_ref[...] + b2_ref[...])
        hscr[pl.ds(i * BR, BR), :] = h

        @pl.when(i == 0)
        def _():
            acc1[...] = jnp.zeros_like(acc1)
            acc2[...] = jnp.zeros_like(acc2)

        acc1[...] += jnp.sum(h, axis=0, keepdims=True)
        acc2[...] += jnp.sum(h * h, axis=0, keepdims=True)

    @pl.when(p == 1)
    def _():
        s1 = acc1[...] * (1.0 / N)
        mms = s1 * gms_ref[...]
        var = acc2[...] * (1.0 / N) - 2.0 * mms * s1 + mms * mms
        scale = gw_ref[...] * lax.rsqrt(var + EPS)
        h = hscr[pl.ds(i * BR, BR), :]
        o_ref[...] = jnp.maximum(scale * (h - mms) + gb_ref[...], 0.0)


_final = pl.pallas_call(
    _fin_body,
    grid=(2, GRID),
    in_specs=[pl.BlockSpec((BR, 1), lambda p, i: (i, 0)),
              pl.BlockSpec((1, BR, DH), lambda p, i: (0, i, 0)),
              pl.BlockSpec((1, BR, DH), lambda p, i: (1, i, 0)),
              pl.BlockSpec((1, BR, DH), lambda p, i: (0, i, 0)),
              pl.BlockSpec((1, BR, DH), lambda p, i: (1, i, 0)),
              pl.BlockSpec((1, D_OUT), lambda p, i: (0, 0)),
              pl.BlockSpec((1, D_OUT), lambda p, i: (0, 0)),
              pl.BlockSpec((1, D_OUT), lambda p, i: (0, 0)),
              pl.BlockSpec((1, D_OUT), lambda p, i: (0, 0))],
    out_specs=pl.BlockSpec((BR, D_OUT), lambda p, i: (i, 0)),
    out_shape=jax.ShapeDtypeStruct((N, D_OUT), jnp.float32),
    scratch_shapes=[pltpu.VMEM((N, D_OUT), jnp.float32),
                    pltpu.VMEM((1, D_OUT), jnp.float32),
                    pltpu.VMEM((1, D_OUT), jnp.float32)],
)


def kernel(x, edge_index, W1, b1, gn1_w, gn1_b, gn1_ms, W2, b2,
           gn2_w, gn2_b, gn2_ms):
    pad = EPAD - E
    src2d = jnp.concatenate(
        [edge_index[0], jnp.zeros((pad,), jnp.int32)]).reshape(-1, 128)
    dst2d = jnp.concatenate(
        [edge_index[1], jnp.full((pad,), NPAD - 1, jnp.int32)]).reshape(-1, 128)

    deg = _deg_sc(dst2d).reshape(2, NPAD, 1)
    xs3, dinv = _prep(deg, deg, x)
    p = _agg_sc(xs3.reshape(2 * NPAD, DH), src2d, dst2d)
    zs3 = _layer12(dinv, p, p, xs3, xs3, W1, b1.reshape(1, D_HID),
                   gn1_w.reshape(1, D_HID), gn1_b.reshape(1, D_HID),
                   gn1_ms.reshape(1, D_HID), W2)
    q = _agg_sc(zs3.reshape(2 * NPAD, DH), src2d, dst2d)
    return _final(dinv, q, q, zs3, zs3, b2.reshape(1, D_OUT),
                  gn2_w.reshape(1, D_OUT), gn2_b.reshape(1, D_OUT),
                  gn2_ms.reshape(1, D_OUT))
